# baseline (speedup 1.0000x reference)
import functools

import jax
import jax.numpy as jnp
from jax import lax
from jax.experimental import pallas as pl
from jax.experimental.pallas import tpu as pltpu
from jax.experimental.pallas import tpu_sc as plsc

B = 16384
D = 64
NROWS = 1_000_000
NMAIN = NROWS // 128 * 128
NTAIL = NROWS - NMAIN

_info = plsc.get_sparse_core_info()
_NC, _NS = _info.num_cores, _info.num_subcores
_NW = _NC * _NS
_WAVE = 512
_BLK = 4096
_NWAVES = -(-NROWS // (_NW * _WAVE))
_RANGE = _NWAVES * _WAVE
_OUT_PAD = B + _NW
_WCAP = 4096

_sc_mesh = plsc.VectorSubcoreMesh(core_axis_name="c", subcore_axis_name="s")


@functools.partial(
    pl.kernel,
    mesh=_sc_mesh,
    compiler_params=pltpu.CompilerParams(needs_layout_passes=False),
    out_type=[
        jax.ShapeDtypeStruct((_OUT_PAD, 128), jnp.float32),
        jax.ShapeDtypeStruct((_OUT_PAD, 128), jnp.float32),
    ],
    scratch_types=[
        pltpu.VMEM((B,), jnp.int32),
        pltpu.VMEM((B,), jnp.int32),
        pltpu.VMEM((_WCAP,), jnp.int32),
        pltpu.VMEM((_WCAP,), jnp.int32),
        pltpu.VMEM((D, _WAVE), jnp.float32),
        pltpu.VMEM((D, _WAVE), jnp.float32),
        pltpu.VMEM((16, 128), jnp.float32),
        pltpu.VMEM((16,), jnp.int32),
        pltpu.SemaphoreType.DMA,
        pltpu.SemaphoreType.DMA,
        pltpu.SemaphoreType.DMA,
    ],
)
def _sc_stream_gather(uid_hbm, iid_hbm, tu_hbm, ti_hbm, tu_tail, ti_tail,
                      out_u, out_i,
                      mid_v, mpos_v, wcol_v, wpos_v, buf_a, buf_b,
                      stage_v, pos16_v, sem_a, sem_b, sem_s):
    wid = lax.axis_index("s") * _NC + lax.axis_index("c")
    lo = wid * _RANGE
    iota16 = lax.iota(jnp.int32, 16)
    dummy_row = B + wid
    _U0MAX = NMAIN - _WAVE

    for tab_hbm, tail_hbm, id_hbm, out_hbm in (
        (tu_hbm, tu_tail, uid_hbm, out_u),
        (ti_hbm, ti_tail, iid_hbm, out_i),
    ):
        pltpu.sync_copy(id_hbm, mid_v)

        def scan_body(j, k):
            base = pl.multiple_of(j * 16, 16)
            idv = mid_v[pl.ds(base, 16)]
            m = (idv >= lo) & (idv < lo + _RANGE)
            pop = plsc.all_reduce_population_count(m)[0]
            dest = k + plsc.cumsum(m.astype(jnp.int32)) - 1
            plsc.store_scatter(mid_v, [dest], idv, mask=m)
            plsc.store_scatter(mpos_v, [dest], base + iota16, mask=m)
            return k + pop

        K = lax.fori_loop(0, B // 16, scan_body, jnp.int32(0))
        nmch = (K + 15) // 16

        def rescan_window(wb, hi, u0, off):
            def rescan(j, wc):
                base = pl.multiple_of(j * 16, 16)
                lm = (base + iota16) < K
                midv = mid_v[pl.ds(base, 16)]
                m = lm & (midv >= wb) & (midv < hi)
                pop = plsc.all_reduce_population_count(m)[0]
                mposv = mpos_v[pl.ds(base, 16)]
                dest = wc + plsc.cumsum(m.astype(jnp.int32)) - 1 - off
                sm = m & (dest >= 0) & (dest < _WCAP)
                plsc.store_scatter(wcol_v, [dest], midv - u0, mask=sm)
                plsc.store_scatter(wpos_v, [dest], mposv, mask=sm)
                return wc + pop

            return lax.fori_loop(0, nmch, rescan, jnp.int32(0))

        def extract_chunks(Mw, src_v):
            def chunk(c, _c):
                base = pl.multiple_of(c * 16, 16)
                lm = (base + iota16) < Mw
                col = jnp.where(lm, wcol_v[pl.ds(base, 16)], 0)
                posv = jnp.where(lm, wpos_v[pl.ds(base, 16)], dummy_row)
                pos16_v[...] = posv
                for e in range(D):
                    erow = jnp.full((16,), e, jnp.int32)
                    vals = plsc.load_gather(src_v, [erow, col])
                    plsc.store_scatter(stage_v, [iota16, erow], vals)
                pltpu.async_copy(stage_v, out_hbm.at[pos16_v], sem_s).wait()
                return _c

            lax.fori_loop(0, (Mw + 15) // 16, chunk, jnp.int32(0))

        def process_window(wb, hi, u0, src_v):
            M = rescan_window(wb, hi, u0, jnp.int32(0))
            extract_chunks(jnp.minimum(M, _WCAP), src_v)

            @pl.when(M > _WCAP)
            def _overflow():
                def pass_body(p, _):
                    off = p * _WCAP
                    rescan_window(wb, hi, u0, off)
                    extract_chunks(jnp.minimum(M - off, _WCAP), src_v)
                    return _

                lax.fori_loop(1, (M + _WCAP - 1) // _WCAP, pass_body,
                              jnp.int32(0))

        def fast_wave(wb, hi, u0, k2, src_v):
            def chunk(c, _c):
                base = pl.multiple_of(c * 16, 16)
                lm = (base + iota16) < k2
                bid = wcol_v[pl.ds(base, 16)]
                m = lm & (bid >= wb) & (bid < hi)
                pop = plsc.all_reduce_population_count(m)[0]

                @pl.when(pop > 0)
                def _extract():
                    bpos = wpos_v[pl.ds(base, 16)]
                    col = jnp.where(m, bid - u0, 0)
                    posv = jnp.where(m, bpos, dummy_row)
                    pos16_v[...] = posv
                    for e in range(D):
                        erow = jnp.full((16,), e, jnp.int32)
                        vals = plsc.load_gather(src_v, [erow, col])
                        plsc.store_scatter(stage_v, [iota16, erow], vals)
                    pltpu.async_copy(stage_v, out_hbm.at[pos16_v],
                                     sem_s).wait()

                return _c

            lax.fori_loop(0, (k2 + 15) // 16, chunk, jnp.int32(0))

        def wave_bounds(v):
            wb = lo + v * _WAVE
            u0 = pl.multiple_of(jnp.minimum(wb, _U0MAX), 128)
            hi = jnp.minimum(wb + _WAVE, u0 + _WAVE)
            return wb, hi, u0

        def start_dma(v, buf, sem):
            _, _, u0 = wave_bounds(v)
            pltpu.async_copy(tab_hbm.at[:, pl.ds(u0, _WAVE)], buf, sem)

        def wait_dma(buf, sem):
            pltpu.make_async_copy(
                tab_hbm.at[:, pl.ds(0, _WAVE)], buf, sem).wait()

        def process(v, k2, src_v):
            wb, hi, u0 = wave_bounds(v)

            @pl.when(k2 <= _WCAP)
            def _fast():
                fast_wave(wb, hi, u0, k2, src_v)

            @pl.when(k2 > _WCAP)
            def _slow():
                process_window(wb, hi, u0, src_v)

        def block_refresh(v, k2):
            blo = lo + (v // (_BLK // _WAVE)) * _BLK
            return lax.cond(
                v % (_BLK // _WAVE) == 0,
                lambda: rescan_window(blo, blo + _BLK, jnp.int32(0),
                                      jnp.int32(0)),
                lambda: k2)

        start_dma(jnp.int32(0), buf_a, sem_a)

        @pl.loop(0, _NWAVES, step=2, init_carry=jnp.int32(0))
        def _pipe(g, k2):
            start_dma(g + 1, buf_b, sem_b)
            k2 = block_refresh(g, k2)
            wait_dma(buf_a, sem_a)
            process(g, k2, buf_a)

            @pl.when(g + 2 < _NWAVES)
            def _nxt():
                start_dma(g + 2, buf_a, sem_a)

            k2b = block_refresh(g + 1, k2)
            wait_dma(buf_b, sem_b)
            process(g + 1, k2b, buf_b)
            return k2b

        @pl.when(wid == _NW - 1)
        def _tail():
            pltpu.sync_copy(tail_hbm, buf_a.at[:, pl.ds(0, 128)])
            process_window(jnp.int32(NMAIN), jnp.int32(NROWS),
                           jnp.int32(NMAIN), buf_a)


_BM = 2048


def _mlp_body(u_ref, i_ref, w1u_ref, w1i_ref, b1_ref, w2_ref, b2_ref,
              w3_ref, b3_ref, out_ref):
    x = jnp.dot(u_ref[...][:, :D], w1u_ref[...],
                preferred_element_type=jnp.float32)
    x = x + jnp.dot(i_ref[...][:, :D], w1i_ref[...],
                    preferred_element_type=jnp.float32)
    x = jnp.maximum(x + b1_ref[...], 0.0)
    x = jnp.maximum(
        jnp.dot(x, w2_ref[...], preferred_element_type=jnp.float32)
        + b2_ref[...], 0.0)
    out_ref[...] = jnp.sum(x * w3_ref[...], axis=1) + b3_ref[0, 0]


def _mlp(u, i, w1u_t, w1i_t, b1, w2_t, b2, w3, b3):
    grid = (B // _BM,)
    return pl.pallas_call(
        _mlp_body,
        grid=grid,
        in_specs=[
            pl.BlockSpec((_BM, 128), lambda g: (g, 0)),
            pl.BlockSpec((_BM, 128), lambda g: (g, 0)),
            pl.BlockSpec((D, 128), lambda g: (0, 0)),
            pl.BlockSpec((D, 128), lambda g: (0, 0)),
            pl.BlockSpec((1, 128), lambda g: (0, 0)),
            pl.BlockSpec((128, D), lambda g: (0, 0)),
            pl.BlockSpec((1, D), lambda g: (0, 0)),
            pl.BlockSpec((1, D), lambda g: (0, 0)),
            pl.BlockSpec((1, 1), lambda g: (0, 0)),
        ],
        out_specs=pl.BlockSpec((_BM,), lambda g: (g,)),
        out_shape=jax.ShapeDtypeStruct((B,), jnp.float32),
    )(u, i, w1u_t, w1i_t, b1, w2_t, b2, w3, b3)


def kernel(user_ids, item_ids, user_table, item_table, W1, b1, W2, b2, W3, b3):
    uids = user_ids.astype(jnp.int32)
    iids = item_ids.astype(jnp.int32)
    tu_main = user_table.T
    ti_main = item_table.T
    tu_tail = jnp.pad(user_table[NMAIN:], ((0, 128 - NTAIL), (0, 0))).T
    ti_tail = jnp.pad(item_table[NMAIN:], ((0, 128 - NTAIL), (0, 0))).T
    u_emb, i_emb = _sc_stream_gather(uids, iids, tu_main, ti_main,
                                     tu_tail, ti_tail)
    w1u_t = W1[:, :D].T
    w1i_t = W1[:, D:].T
    out = _mlp(u_emb, i_emb, w1u_t, w1i_t, b1.reshape(1, 128),
               W2.T, b2.reshape(1, D), W3.reshape(1, D), b3.reshape(1, 1))
    return out


# speedup vs baseline: 2.1858x; 2.1858x over previous
import functools

import jax
import jax.numpy as jnp
from jax import lax
from jax.experimental import pallas as pl
from jax.experimental.pallas import tpu as pltpu
from jax.experimental.pallas import tpu_sc as plsc

B = 16384
D = 64
NROWS = 1_000_000
NMAIN = NROWS // 128 * 128
NTAIL = NROWS - NMAIN

_info = plsc.get_sparse_core_info()
_NC, _NS = _info.num_cores, _info.num_subcores
_NW = _NC * _NS
_WAVE = 512
_BLK = 4096
_NWAVES = -(-NROWS // (_NW * _WAVE))
_RANGE = _NWAVES * _WAVE
_OUT_PAD = B + _NW
_WCAP = 4096

_sc_mesh = plsc.VectorSubcoreMesh(core_axis_name="c", subcore_axis_name="s")


@functools.partial(
    pl.kernel,
    mesh=_sc_mesh,
    compiler_params=pltpu.CompilerParams(needs_layout_passes=False),
    out_type=[
        jax.ShapeDtypeStruct((_OUT_PAD, 128), jnp.float32),
        jax.ShapeDtypeStruct((_OUT_PAD, 128), jnp.float32),
    ],
    scratch_types=[
        pltpu.VMEM((B,), jnp.int32),
        pltpu.VMEM((B,), jnp.int32),
        pltpu.VMEM((_WCAP,), jnp.int32),
        pltpu.VMEM((_WCAP,), jnp.int32),
        pltpu.VMEM((D, _WAVE), jnp.float32),
        pltpu.VMEM((D, _WAVE), jnp.float32),
        pltpu.VMEM((16, 128), jnp.float32),
        pltpu.VMEM((16,), jnp.int32),
        pltpu.SemaphoreType.DMA,
        pltpu.SemaphoreType.DMA,
        pltpu.SemaphoreType.DMA,
    ],
)
def _sc_stream_gather(uid_hbm, iid_hbm, tu_hbm, ti_hbm, tu_tail, ti_tail,
                      out_u, out_i,
                      mid_v, mpos_v, wcol_v, wpos_v, buf_a, buf_b,
                      stage_v, pos16_v, sem_a, sem_b, sem_s):
    wid = lax.axis_index("s") * _NC + lax.axis_index("c")
    lo = wid * _RANGE
    iota16 = lax.iota(jnp.int32, 16)
    dummy_row = B + wid
    _U0MAX = NMAIN - _WAVE

    for tab_hbm, tail_hbm, id_hbm, out_hbm in (
        (tu_hbm, tu_tail, uid_hbm, out_u),
        (ti_hbm, ti_tail, iid_hbm, out_i),
    ):
        pltpu.sync_copy(id_hbm, mid_v)

        def scan_body(j, k):
            base = pl.multiple_of(j * 16, 16)
            idv = mid_v[pl.ds(base, 16)]
            m = (idv >= lo) & (idv < lo + _RANGE)
            pop = plsc.all_reduce_population_count(m)[0]
            dest = k + plsc.cumsum(m.astype(jnp.int32)) - 1
            plsc.store_scatter(mid_v, [dest], idv, mask=m)
            plsc.store_scatter(mpos_v, [dest], base + iota16, mask=m)
            return k + pop

        K = lax.fori_loop(0, B // 16, scan_body, jnp.int32(0))
        nmch = (K + 15) // 16

        def rescan_window(wb, hi, u0, off):
            def rescan(j, wc):
                base = pl.multiple_of(j * 16, 16)
                lm = (base + iota16) < K
                midv = mid_v[pl.ds(base, 16)]
                m = lm & (midv >= wb) & (midv < hi)
                pop = plsc.all_reduce_population_count(m)[0]
                mposv = mpos_v[pl.ds(base, 16)]
                dest = wc + plsc.cumsum(m.astype(jnp.int32)) - 1 - off
                sm = m & (dest >= 0) & (dest < _WCAP)
                plsc.store_scatter(wcol_v, [dest], midv - u0, mask=sm)
                plsc.store_scatter(wpos_v, [dest], mposv, mask=sm)
                return wc + pop

            return lax.fori_loop(0, nmch, rescan, jnp.int32(0))

        def extract_chunks(Mw, src_v):
            def chunk(c, _c):
                base = pl.multiple_of(c * 16, 16)
                lm = (base + iota16) < Mw
                col = jnp.where(lm, wcol_v[pl.ds(base, 16)], 0)
                posv = jnp.where(lm, wpos_v[pl.ds(base, 16)], dummy_row)
                pos16_v[...] = posv
                for e in range(D):
                    erow = jnp.full((16,), e, jnp.int32)
                    vals = plsc.load_gather(src_v, [erow, col])
                    plsc.store_scatter(stage_v, [iota16, erow], vals)
                pltpu.async_copy(stage_v, out_hbm.at[pos16_v], sem_s).wait()
                return _c

            lax.fori_loop(0, (Mw + 15) // 16, chunk, jnp.int32(0))

        def process_window(wb, hi, u0, src_v):
            M = rescan_window(wb, hi, u0, jnp.int32(0))
            extract_chunks(jnp.minimum(M, _WCAP), src_v)

            @pl.when(M > _WCAP)
            def _overflow():
                def pass_body(p, _):
                    off = p * _WCAP
                    rescan_window(wb, hi, u0, off)
                    extract_chunks(jnp.minimum(M - off, _WCAP), src_v)
                    return _

                lax.fori_loop(1, (M + _WCAP - 1) // _WCAP, pass_body,
                              jnp.int32(0))

        def wave_bounds(v):
            wb = lo + v * _WAVE
            u0 = pl.multiple_of(jnp.minimum(wb, _U0MAX), 128)
            hi = jnp.minimum(wb + _WAVE, u0 + _WAVE)
            return wb, hi, u0

        def start_dma(v, buf, sem):
            _, _, u0 = wave_bounds(v)
            pltpu.async_copy(tab_hbm.at[:, pl.ds(u0, _WAVE)], buf, sem)

        def wait_dma(buf, sem):
            pltpu.make_async_copy(
                tab_hbm.at[:, pl.ds(0, _WAVE)], buf, sem).wait()

        def process(v, src_v):
            wb, hi, u0 = wave_bounds(v)
            process_window(wb, hi, u0, src_v)

        start_dma(jnp.int32(0), buf_a, sem_a)

        @pl.loop(0, _NWAVES, step=2)
        def _pipe(g):
            start_dma(g + 1, buf_b, sem_b)
            wait_dma(buf_a, sem_a)
            process(g, buf_a)

            @pl.when(g + 2 < _NWAVES)
            def _nxt():
                start_dma(g + 2, buf_a, sem_a)

            wait_dma(buf_b, sem_b)
            process(g + 1, buf_b)

        @pl.when(wid == _NW - 1)
        def _tail():
            pltpu.sync_copy(tail_hbm, buf_a.at[:, pl.ds(0, 128)])
            process_window(jnp.int32(NMAIN), jnp.int32(NROWS),
                           jnp.int32(NMAIN), buf_a)


_BM = 2048


def _mlp_body(u_ref, i_ref, w1u_ref, w1i_ref, b1_ref, w2_ref, b2_ref,
              w3_ref, b3_ref, out_ref):
    x = jnp.dot(u_ref[...][:, :D], w1u_ref[...],
                preferred_element_type=jnp.float32)
    x = x + jnp.dot(i_ref[...][:, :D], w1i_ref[...],
                    preferred_element_type=jnp.float32)
    x = jnp.maximum(x + b1_ref[...], 0.0)
    x = jnp.maximum(
        jnp.dot(x, w2_ref[...], preferred_element_type=jnp.float32)
        + b2_ref[...], 0.0)
    out_ref[...] = jnp.sum(x * w3_ref[...], axis=1) + b3_ref[0, 0]


def _mlp(u, i, w1u_t, w1i_t, b1, w2_t, b2, w3, b3):
    grid = (B // _BM,)
    return pl.pallas_call(
        _mlp_body,
        grid=grid,
        in_specs=[
            pl.BlockSpec((_BM, 128), lambda g: (g, 0)),
            pl.BlockSpec((_BM, 128), lambda g: (g, 0)),
            pl.BlockSpec((D, 128), lambda g: (0, 0)),
            pl.BlockSpec((D, 128), lambda g: (0, 0)),
            pl.BlockSpec((1, 128), lambda g: (0, 0)),
            pl.BlockSpec((128, D), lambda g: (0, 0)),
            pl.BlockSpec((1, D), lambda g: (0, 0)),
            pl.BlockSpec((1, D), lambda g: (0, 0)),
            pl.BlockSpec((1, 1), lambda g: (0, 0)),
        ],
        out_specs=pl.BlockSpec((_BM,), lambda g: (g,)),
        out_shape=jax.ShapeDtypeStruct((B,), jnp.float32),
    )(u, i, w1u_t, w1i_t, b1, w2_t, b2, w3, b3)


def kernel(user_ids, item_ids, user_table, item_table, W1, b1, W2, b2, W3, b3):
    uids = user_ids.astype(jnp.int32)
    iids = item_ids.astype(jnp.int32)
    tu_main = user_table.T
    ti_main = item_table.T
    tu_tail = jnp.pad(user_table[NMAIN:], ((0, 128 - NTAIL), (0, 0))).T
    ti_tail = jnp.pad(item_table[NMAIN:], ((0, 128 - NTAIL), (0, 0))).T
    u_emb, i_emb = _sc_stream_gather(uids, iids, tu_main, ti_main,
                                     tu_tail, ti_tail)
    w1u_t = W1[:, :D].T
    w1i_t = W1[:, D:].T
    out = _mlp(u_emb, i_emb, w1u_t, w1i_t, b1.reshape(1, 128),
               W2.T, b2.reshape(1, D), W3.reshape(1, D), b3.reshape(1, 1))
    return out


# speedup vs baseline: 2.4843x; 1.1366x over previous
import functools

import jax
import jax.numpy as jnp
from jax import lax
from jax.experimental import pallas as pl
from jax.experimental.pallas import tpu as pltpu
from jax.experimental.pallas import tpu_sc as plsc

B = 16384
D = 64
NROWS = 1_000_000
NMAIN = NROWS // 128 * 128
NTAIL = NROWS - NMAIN

_info = plsc.get_sparse_core_info()
_NC, _NS = _info.num_cores, _info.num_subcores
_NW = _NC * _NS
_WAVE = 640
_NWAVES = -(-NROWS // (_NW * _WAVE))
_NWAVES += _NWAVES % 2
_RANGE = _NWAVES * _WAVE
_OUT_PAD = B + _NW
_WCAP = 4096

_sc_mesh = plsc.VectorSubcoreMesh(core_axis_name="c", subcore_axis_name="s")


@functools.partial(
    pl.kernel,
    mesh=_sc_mesh,
    compiler_params=pltpu.CompilerParams(needs_layout_passes=False),
    out_type=[
        jax.ShapeDtypeStruct((_OUT_PAD, 128), jnp.float32),
        jax.ShapeDtypeStruct((_OUT_PAD, 128), jnp.float32),
    ],
    scratch_types=[
        pltpu.VMEM((B,), jnp.int32),
        pltpu.VMEM((B,), jnp.int32),
        pltpu.VMEM((_WCAP,), jnp.int32),
        pltpu.VMEM((_WCAP,), jnp.int32),
        pltpu.VMEM((D, _WAVE), jnp.float32),
        pltpu.VMEM((D, _WAVE), jnp.float32),
        pltpu.VMEM((16, 128), jnp.float32),
        pltpu.VMEM((16,), jnp.int32),
        pltpu.SemaphoreType.DMA,
        pltpu.SemaphoreType.DMA,
        pltpu.SemaphoreType.DMA,
    ],
)
def _sc_stream_gather(uid_hbm, iid_hbm, tu_hbm, ti_hbm, tu_tail, ti_tail,
                      out_u, out_i,
                      mid_v, mpos_v, wcol_v, wpos_v, buf_a, buf_b,
                      stage_v, pos16_v, sem_a, sem_b, sem_s):
    wid = lax.axis_index("s") * _NC + lax.axis_index("c")
    lo = wid * _RANGE
    iota16 = lax.iota(jnp.int32, 16)
    dummy_row = B + wid
    _U0MAX = NMAIN - _WAVE

    for tab_hbm, tail_hbm, id_hbm, out_hbm in (
        (tu_hbm, tu_tail, uid_hbm, out_u),
        (ti_hbm, ti_tail, iid_hbm, out_i),
    ):
        pltpu.sync_copy(id_hbm, mid_v)

        def scan_body(j, k):
            base = pl.multiple_of(j * 16, 16)
            idv = mid_v[pl.ds(base, 16)]
            m = (idv >= lo) & (idv < lo + _RANGE)
            pop = plsc.all_reduce_population_count(m)[0]
            dest = k + plsc.cumsum(m.astype(jnp.int32)) - 1
            plsc.store_scatter(mid_v, [dest], idv, mask=m)
            plsc.store_scatter(mpos_v, [dest], base + iota16, mask=m)
            return k + pop

        K = lax.fori_loop(0, B // 16, scan_body, jnp.int32(0))
        nmch = (K + 15) // 16

        def rescan_window(wb, hi, u0, off):
            def rescan(j, wc):
                base = pl.multiple_of(j * 16, 16)
                lm = (base + iota16) < K
                midv = mid_v[pl.ds(base, 16)]
                m = lm & (midv >= wb) & (midv < hi)
                pop = plsc.all_reduce_population_count(m)[0]
                mposv = mpos_v[pl.ds(base, 16)]
                dest = wc + plsc.cumsum(m.astype(jnp.int32)) - 1 - off
                sm = m & (dest >= 0) & (dest < _WCAP)
                plsc.store_scatter(wcol_v, [dest], midv - u0, mask=sm)
                plsc.store_scatter(wpos_v, [dest], mposv, mask=sm)
                return wc + pop

            return lax.fori_loop(0, nmch, rescan, jnp.int32(0))

        def extract_chunks(Mw, src_v):
            def chunk(c, _c):
                base = pl.multiple_of(c * 16, 16)
                lm = (base + iota16) < Mw
                col = jnp.where(lm, wcol_v[pl.ds(base, 16)], 0)
                posv = jnp.where(lm, wpos_v[pl.ds(base, 16)], dummy_row)
                pos16_v[...] = posv
                for e in range(D):
                    erow = jnp.full((16,), e, jnp.int32)
                    vals = plsc.load_gather(src_v, [erow, col])
                    plsc.store_scatter(stage_v, [iota16, erow], vals)
                pltpu.async_copy(stage_v, out_hbm.at[pos16_v], sem_s).wait()
                return _c

            lax.fori_loop(0, (Mw + 15) // 16, chunk, jnp.int32(0))

        def process_window(wb, hi, u0, src_v):
            M = rescan_window(wb, hi, u0, jnp.int32(0))
            extract_chunks(jnp.minimum(M, _WCAP), src_v)

            @pl.when(M > _WCAP)
            def _overflow():
                def pass_body(p, _):
                    off = p * _WCAP
                    rescan_window(wb, hi, u0, off)
                    extract_chunks(jnp.minimum(M - off, _WCAP), src_v)
                    return _

                lax.fori_loop(1, (M + _WCAP - 1) // _WCAP, pass_body,
                              jnp.int32(0))

        def wave_bounds(v):
            wb = lo + v * _WAVE
            u0 = pl.multiple_of(jnp.minimum(wb, _U0MAX), 128)
            hi = jnp.minimum(wb + _WAVE, u0 + _WAVE)
            return wb, hi, u0

        def start_dma(v, buf, sem):
            _, _, u0 = wave_bounds(v)
            pltpu.async_copy(tab_hbm.at[:, pl.ds(u0, _WAVE)], buf, sem)

        def wait_dma(buf, sem):
            pltpu.make_async_copy(
                tab_hbm.at[:, pl.ds(0, _WAVE)], buf, sem).wait()

        def process(v, src_v):
            wb, hi, u0 = wave_bounds(v)
            process_window(wb, hi, u0, src_v)

        start_dma(jnp.int32(0), buf_a, sem_a)

        @pl.loop(0, _NWAVES, step=2)
        def _pipe(g):
            start_dma(g + 1, buf_b, sem_b)
            wait_dma(buf_a, sem_a)
            process(g, buf_a)

            @pl.when(g + 2 < _NWAVES)
            def _nxt():
                start_dma(g + 2, buf_a, sem_a)

            wait_dma(buf_b, sem_b)
            process(g + 1, buf_b)

        @pl.when(wid == _NW - 1)
        def _tail():
            pltpu.sync_copy(tail_hbm, buf_a.at[:, pl.ds(0, 128)])
            process_window(jnp.int32(NMAIN), jnp.int32(NROWS),
                           jnp.int32(NMAIN), buf_a)


_BM = 2048


def _mlp_body(u_ref, i_ref, w1u_ref, w1i_ref, b1_ref, w2_ref, b2_ref,
              w3_ref, b3_ref, out_ref):
    x = jnp.dot(u_ref[...][:, :D], w1u_ref[...],
                preferred_element_type=jnp.float32)
    x = x + jnp.dot(i_ref[...][:, :D], w1i_ref[...],
                    preferred_element_type=jnp.float32)
    x = jnp.maximum(x + b1_ref[...], 0.0)
    x = jnp.maximum(
        jnp.dot(x, w2_ref[...], preferred_element_type=jnp.float32)
        + b2_ref[...], 0.0)
    out_ref[...] = jnp.sum(x * w3_ref[...], axis=1) + b3_ref[0, 0]


def _mlp(u, i, w1u_t, w1i_t, b1, w2_t, b2, w3, b3):
    grid = (B // _BM,)
    return pl.pallas_call(
        _mlp_body,
        grid=grid,
        in_specs=[
            pl.BlockSpec((_BM, 128), lambda g: (g, 0)),
            pl.BlockSpec((_BM, 128), lambda g: (g, 0)),
            pl.BlockSpec((D, 128), lambda g: (0, 0)),
            pl.BlockSpec((D, 128), lambda g: (0, 0)),
            pl.BlockSpec((1, 128), lambda g: (0, 0)),
            pl.BlockSpec((128, D), lambda g: (0, 0)),
            pl.BlockSpec((1, D), lambda g: (0, 0)),
            pl.BlockSpec((1, D), lambda g: (0, 0)),
            pl.BlockSpec((1, 1), lambda g: (0, 0)),
        ],
        out_specs=pl.BlockSpec((_BM,), lambda g: (g,)),
        out_shape=jax.ShapeDtypeStruct((B,), jnp.float32),
    )(u, i, w1u_t, w1i_t, b1, w2_t, b2, w3, b3)


def kernel(user_ids, item_ids, user_table, item_table, W1, b1, W2, b2, W3, b3):
    uids = user_ids.astype(jnp.int32)
    iids = item_ids.astype(jnp.int32)
    tu_main = user_table.T
    ti_main = item_table.T
    tu_tail = jnp.pad(user_table[NMAIN:], ((0, 128 - NTAIL), (0, 0))).T
    ti_tail = jnp.pad(item_table[NMAIN:], ((0, 128 - NTAIL), (0, 0))).T
    u_emb, i_emb = _sc_stream_gather(uids, iids, tu_main, ti_main,
                                     tu_tail, ti_tail)
    w1u_t = W1[:, :D].T
    w1i_t = W1[:, D:].T
    out = _mlp(u_emb, i_emb, w1u_t, w1i_t, b1.reshape(1, 128),
               W2.T, b2.reshape(1, D), W3.reshape(1, D), b3.reshape(1, 1))
    return out


# speedup vs baseline: 2.5366x; 1.0210x over previous
import functools

import jax
import jax.numpy as jnp
from jax import lax
from jax.experimental import pallas as pl
from jax.experimental.pallas import tpu as pltpu
from jax.experimental.pallas import tpu_sc as plsc

B = 16384
D = 64
NROWS = 1_000_000
NMAIN = NROWS // 128 * 128
NTAIL = NROWS - NMAIN

_info = plsc.get_sparse_core_info()
_NC, _NS = _info.num_cores, _info.num_subcores
_NW = _NC * _NS
_WAVE = 640
_NWAVES = -(-NROWS // (_NW * _WAVE))
_NWAVES += _NWAVES % 2
_RANGE = _NWAVES * _WAVE
_OUT_PAD = B + _NW
_WCAP = 4096

_sc_mesh = plsc.VectorSubcoreMesh(core_axis_name="c", subcore_axis_name="s")


@functools.partial(
    pl.kernel,
    mesh=_sc_mesh,
    compiler_params=pltpu.CompilerParams(needs_layout_passes=False),
    out_type=[
        jax.ShapeDtypeStruct((_OUT_PAD, 128), jnp.float32),
        jax.ShapeDtypeStruct((_OUT_PAD, 128), jnp.float32),
    ],
    scratch_types=[
        pltpu.VMEM((B,), jnp.int32),
        pltpu.VMEM((B,), jnp.int32),
        pltpu.VMEM((_WCAP,), jnp.int32),
        pltpu.VMEM((_WCAP,), jnp.int32),
        pltpu.VMEM((D, _WAVE), jnp.float32),
        pltpu.VMEM((D, _WAVE), jnp.float32),
        pltpu.VMEM((16, 128), jnp.float32),
        pltpu.VMEM((16,), jnp.int32),
        pltpu.SemaphoreType.DMA,
        pltpu.SemaphoreType.DMA,
        pltpu.SemaphoreType.DMA,
    ],
)
def _sc_stream_gather(uid_hbm, iid_hbm, tu_hbm, ti_hbm, tu_tail, ti_tail,
                      out_u, out_i,
                      mid_v, mpos_v, wcol_v, wpos_v, buf_a, buf_b,
                      stage_v, pos16_v, sem_a, sem_b, sem_s):
    wid = lax.axis_index("s") * _NC + lax.axis_index("c")
    lo = wid * _RANGE
    iota16 = lax.iota(jnp.int32, 16)
    dummy_row = B + wid
    _U0MAX = NMAIN - _WAVE

    for tab_hbm, tail_hbm, id_hbm, out_hbm in (
        (tu_hbm, tu_tail, uid_hbm, out_u),
        (ti_hbm, ti_tail, iid_hbm, out_i),
    ):
        pltpu.sync_copy(id_hbm, mid_v)

        def scan_body(j, k):
            base = pl.multiple_of(j * 16, 16)
            idv = mid_v[pl.ds(base, 16)]
            m = (idv >= lo) & (idv < lo + _RANGE)
            pop = plsc.all_reduce_population_count(m)[0]
            dest = k + plsc.cumsum(m.astype(jnp.int32)) - 1
            plsc.store_scatter(mid_v, [dest], idv, mask=m)
            plsc.store_scatter(mpos_v, [dest], base + iota16, mask=m)
            return k + pop

        K = lax.fori_loop(0, B // 16, scan_body, jnp.int32(0))
        nmch = (K + 15) // 16

        def rescan_window(wb, hi, u0, off):
            def rescan(j, wc):
                base = pl.multiple_of(j * 16, 16)
                lm = (base + iota16) < K
                midv = mid_v[pl.ds(base, 16)]
                m = lm & (midv >= wb) & (midv < hi)
                pop = plsc.all_reduce_population_count(m)[0]
                mposv = mpos_v[pl.ds(base, 16)]
                dest = wc + plsc.cumsum(m.astype(jnp.int32)) - 1 - off
                sm = m & (dest >= 0) & (dest < _WCAP)
                plsc.store_scatter(wcol_v, [dest], midv - u0, mask=sm)
                plsc.store_scatter(wpos_v, [dest], mposv, mask=sm)
                return wc + pop

            return lax.fori_loop(0, nmch, rescan, jnp.int32(0))

        def extract_chunks(Mw, src_v):
            def chunk(c, _c):
                base = pl.multiple_of(c * 16, 16)
                lm = (base + iota16) < Mw
                col = jnp.where(lm, wcol_v[pl.ds(base, 16)], 0)
                posv = jnp.where(lm, wpos_v[pl.ds(base, 16)], dummy_row)
                pos16_v[...] = posv
                for e in range(D):
                    erow = jnp.full((16,), e, jnp.int32)
                    vals = plsc.load_gather(src_v, [erow, col])
                    plsc.store_scatter(stage_v, [iota16, erow], vals)
                pltpu.async_copy(stage_v, out_hbm.at[pos16_v], sem_s).wait()
                return _c

            lax.fori_loop(0, (Mw + 15) // 16, chunk, jnp.int32(0))

        def extract_window(wb, hi, u0, M, src_v):
            extract_chunks(jnp.minimum(M, _WCAP), src_v)

            @pl.when(M > _WCAP)
            def _overflow():
                def pass_body(p, _):
                    off = p * _WCAP
                    rescan_window(wb, hi, u0, off)
                    extract_chunks(jnp.minimum(M - off, _WCAP), src_v)
                    return _

                lax.fori_loop(1, (M + _WCAP - 1) // _WCAP, pass_body,
                              jnp.int32(0))

        def wave_bounds(v):
            wb = lo + v * _WAVE
            u0 = pl.multiple_of(jnp.minimum(wb, _U0MAX), 128)
            hi = jnp.minimum(wb + _WAVE, u0 + _WAVE)
            return wb, hi, u0

        def start_dma(v, buf, sem):
            _, _, u0 = wave_bounds(v)
            pltpu.async_copy(tab_hbm.at[:, pl.ds(u0, _WAVE)], buf, sem)

        def wait_dma(buf, sem):
            pltpu.make_async_copy(
                tab_hbm.at[:, pl.ds(0, _WAVE)], buf, sem).wait()

        def process_pre(v):
            wb, hi, u0 = wave_bounds(v)
            return rescan_window(wb, hi, u0, jnp.int32(0))

        def process_post(v, M, src_v):
            wb, hi, u0 = wave_bounds(v)
            extract_window(wb, hi, u0, M, src_v)

        start_dma(jnp.int32(0), buf_a, sem_a)

        @pl.loop(0, _NWAVES, step=2)
        def _pipe(g):
            start_dma(g + 1, buf_b, sem_b)
            M0 = process_pre(g)
            wait_dma(buf_a, sem_a)
            process_post(g, M0, buf_a)

            @pl.when(g + 2 < _NWAVES)
            def _nxt():
                start_dma(g + 2, buf_a, sem_a)

            M1 = process_pre(g + 1)
            wait_dma(buf_b, sem_b)
            process_post(g + 1, M1, buf_b)

        @pl.when(wid == _NW - 1)
        def _tail():
            pltpu.sync_copy(tail_hbm, buf_a.at[:, pl.ds(0, 128)])
            Mt = rescan_window(jnp.int32(NMAIN), jnp.int32(NROWS),
                               jnp.int32(NMAIN), jnp.int32(0))
            extract_window(jnp.int32(NMAIN), jnp.int32(NROWS),
                           jnp.int32(NMAIN), Mt, buf_a)


_BM = 2048


def _mlp_body(u_ref, i_ref, w1u_ref, w1i_ref, b1_ref, w2_ref, b2_ref,
              w3_ref, b3_ref, out_ref):
    x = jnp.dot(u_ref[...][:, :D], w1u_ref[...],
                preferred_element_type=jnp.float32)
    x = x + jnp.dot(i_ref[...][:, :D], w1i_ref[...],
                    preferred_element_type=jnp.float32)
    x = jnp.maximum(x + b1_ref[...], 0.0)
    x = jnp.maximum(
        jnp.dot(x, w2_ref[...], preferred_element_type=jnp.float32)
        + b2_ref[...], 0.0)
    out_ref[...] = jnp.sum(x * w3_ref[...], axis=1) + b3_ref[0, 0]


def _mlp(u, i, w1u_t, w1i_t, b1, w2_t, b2, w3, b3):
    grid = (B // _BM,)
    return pl.pallas_call(
        _mlp_body,
        grid=grid,
        in_specs=[
            pl.BlockSpec((_BM, 128), lambda g: (g, 0)),
            pl.BlockSpec((_BM, 128), lambda g: (g, 0)),
            pl.BlockSpec((D, 128), lambda g: (0, 0)),
            pl.BlockSpec((D, 128), lambda g: (0, 0)),
            pl.BlockSpec((1, 128), lambda g: (0, 0)),
            pl.BlockSpec((128, D), lambda g: (0, 0)),
            pl.BlockSpec((1, D), lambda g: (0, 0)),
            pl.BlockSpec((1, D), lambda g: (0, 0)),
            pl.BlockSpec((1, 1), lambda g: (0, 0)),
        ],
        out_specs=pl.BlockSpec((_BM,), lambda g: (g,)),
        out_shape=jax.ShapeDtypeStruct((B,), jnp.float32),
    )(u, i, w1u_t, w1i_t, b1, w2_t, b2, w3, b3)


def kernel(user_ids, item_ids, user_table, item_table, W1, b1, W2, b2, W3, b3):
    uids = user_ids.astype(jnp.int32)
    iids = item_ids.astype(jnp.int32)
    tu_main = user_table.T
    ti_main = item_table.T
    tu_tail = jnp.pad(user_table[NMAIN:], ((0, 128 - NTAIL), (0, 0))).T
    ti_tail = jnp.pad(item_table[NMAIN:], ((0, 128 - NTAIL), (0, 0))).T
    u_emb, i_emb = _sc_stream_gather(uids, iids, tu_main, ti_main,
                                     tu_tail, ti_tail)
    w1u_t = W1[:, :D].T
    w1i_t = W1[:, D:].T
    out = _mlp(u_emb, i_emb, w1u_t, w1i_t, b1.reshape(1, 128),
               W2.T, b2.reshape(1, D), W3.reshape(1, D), b3.reshape(1, 1))
    return out
